# baseline (speedup 1.0000x reference)
import functools

import jax
import jax.numpy as jnp
from jax import lax
from jax.experimental import pallas as pl
from jax.experimental.pallas import tpu as pltpu
from jax.experimental.pallas import tpu_sc as plsc

N_WORKERS = 32
BLOCKS_PER_CHUNK = 2
IDX_PER_CHUNK = BLOCKS_PER_CHUNK * 8 * 128
NBUF = 2


def _render(idx_blk, feats, n_blocks, n_bh):
  C = feats.shape[1]
  blocks_per_w = n_blocks // N_WORKERS
  chunks_per_w = blocks_per_w // BLOCKS_PER_CHUNK

  mesh = plsc.VectorSubcoreMesh(core_axis_name="c", subcore_axis_name="s")

  @functools.partial(
      pl.kernel,
      mesh=mesh,
      out_type=jax.ShapeDtypeStruct((n_bh, 2, 4, 1024), jnp.float32),
      compiler_params=pltpu.CompilerParams(
          use_tc_tiling_on_sc=False, needs_layout_passes=False),
      scratch_types=(
          [pltpu.VMEM((2 * BLOCKS_PER_CHUNK, 8, 128), jnp.int32)] * NBUF
          + [pltpu.VMEM((IDX_PER_CHUNK, C), jnp.float32)] * NBUF
          + [pltpu.VMEM((BLOCKS_PER_CHUNK * 16 * 128,), jnp.float32)]
          + [pltpu.SemaphoreType.DMA] * (2 * NBUF)
      ),
  )
  def k(idx_hbm, feat_hbm, out_hbm, idx_v0, idx_v1, rows_v0, rows_v1,
        out_v, sem0, sem1, isem0, isem1):
    wid = lax.axis_index("s") * 2 + lax.axis_index("c")
    blk0 = wid * blocks_per_w
    idx_bufs = (idx_v0, idx_v1)
    rows_bufs = (rows_v0, rows_v1)
    sems = (sem0, sem1)
    isems = (isem0, isem1)
    iota = lax.iota(jnp.int32, 16)

    def idx_copy(g2, islot):
      pltpu.async_copy(
          idx_hbm.at[pl.ds(blk0 + g2 * 2 * BLOCKS_PER_CHUNK,
                           2 * BLOCKS_PER_CHUNK)],
          idx_bufs[islot], isems[islot])

    def idx_wait(g2, islot):
      pltpu.make_async_copy(
          idx_hbm.at[pl.ds(blk0 + g2 * 2 * BLOCKS_PER_CHUNK,
                           2 * BLOCKS_PER_CHUNK)],
          idx_bufs[islot], isems[islot]).wait()

    def fire(g, slot, islot, half):
      idx_v = idx_bufs[islot]
      rows_v = rows_bufs[slot]
      for b2 in range(BLOCKS_PER_CHUNK):
        for kk in range(8):
          pltpu.async_copy(
              feat_hbm.at[idx_v.at[half * BLOCKS_PER_CHUNK + b2, kk]],
              rows_v.at[pl.ds((b2 * 8 + kk) * 128, 128)],
              sems[slot])

    def drain(slot, islot, half):
      for b2 in range(BLOCKS_PER_CHUNK):
        for kk in range(8):
          pltpu.make_async_copy(
              feat_hbm.at[idx_bufs[islot].at[half * BLOCKS_PER_CHUNK + b2,
                                             kk]],
              rows_bufs[slot].at[pl.ds((b2 * 8 + kk) * 128, 128)],
              sems[slot]).wait()

    def reduce_store(g, slot):
      rows_v = rows_bufs[slot]
      def tree(l):
        s0 = l[0] + l[1]
        s1 = l[2] + l[3]
        s2 = l[4] + l[5]
        s3 = l[6] + l[7]
        return ((s0 + s1) + (s2 + s3)) * 0.125

      for b2 in range(BLOCKS_PER_CHUNK):
        c_off = iota * 128 + b2 * 2048

        def body(w, _b2=b2, _c_off=c_off):
          P = 4
          loads = []
          accs = [None] * P
          for j in range(P):
            base = _b2 * 1024 + w + j
            loads.append([rows_v[base + 128 * kk] for kk in range(8)])
            if j >= 1:
              accs[j - 1] = tree(loads[j - 1])
          accs[P - 1] = tree(loads[P - 1])
          for j in range(P):
            plsc.store_scatter(out_v, [_c_off + (w + j)], accs[j])

        plsc.parallel_loop(0, 128, 4, unroll=4)(body)

      for b2 in range(BLOCKS_PER_CHUNK):
        blk = blk0 + g * BLOCKS_PER_CHUNK + b2
        bh = blk // 4
        wt = blk % 4
        for ct in range(2):
          pltpu.sync_copy(out_v.at[pl.ds(b2 * 2048 + ct * 1024, 1024)],
                          out_hbm.at[bh, ct, wt])

    n_pairs = chunks_per_w // 2
    idx_copy(0, 0)
    idx_wait(0, 0)
    idx_copy(1, 1)
    fire(0, 0, 0, 0)

    def step(g4, _):
      g = g4 * 4
      fire(g + 1, 1, 0, 1)
      drain(0, 0, 0)
      reduce_store(g, 0)
      idx_wait(2 * g4 + 1, 1)
      fire(g + 2, 0, 1, 0)
      drain(1, 0, 1)
      reduce_store(g + 1, 1)

      @pl.when(2 * g4 + 2 < n_pairs)
      def _():
        idx_copy(2 * g4 + 2, 0)

      fire(g + 3, 1, 1, 1)
      drain(0, 1, 0)
      reduce_store(g + 2, 0)

      @pl.when(g + 4 < chunks_per_w)
      def _():
        idx_wait(2 * g4 + 2, 0)
        fire(g + 4, 0, 0, 0)

      drain(1, 1, 1)
      reduce_store(g + 3, 1)

      @pl.when(2 * g4 + 3 < n_pairs)
      def _():
        idx_copy(2 * g4 + 3, 1)

      return ()

    lax.fori_loop(0, chunks_per_w // 4, step, ())

  return k(idx_blk, feats)


_FMT_PBLK = 4096


def _format_features(ftT, P, C):
  n_blk = (P + _FMT_PBLK - 1) // _FMT_PBLK
  out_rows = n_blk * (_FMT_PBLK * C // 128)

  def body(x_ref, o_ref):
    n = _FMT_PBLK * C // 128
    y3 = jnp.swapaxes(x_ref[...], 0, 1).reshape(n, 128 // C, C)
    o_ref[...] = jnp.concatenate(
        [y3[:, cs, :] for cs in range(128 // C)], axis=1)

  return pl.pallas_call(
      body,
      grid=(n_blk,),
      in_specs=[pl.BlockSpec((C, _FMT_PBLK), lambda i: (0, i))],
      out_specs=pl.BlockSpec((_FMT_PBLK * C // 128, 128), lambda i: (i, 0)),
      out_shape=jax.ShapeDtypeStruct((out_rows, 128), jnp.float32),
  )(ftT)


def kernel(idx, dists, features):
  del dists
  B, H, W, K = idx.shape
  P, C = features.shape
  n_bh = B * H
  n_blocks = n_bh * (W // 128)
  idx_blk = (idx.astype(jnp.int32)
             .reshape(B, H, W // 128, 128, K)
             .transpose(0, 1, 2, 4, 3)
             .reshape(n_blocks, K, 128))
  feats_lin = _format_features(features.T, P, C)
  feats_lin = feats_lin.reshape(feats_lin.shape[0] * 128 // C, C)
  out5 = _render(idx_blk, feats_lin, n_blocks, n_bh)
  out = (out5.reshape(B, H, 2, W // 128, 8, 128)
         .transpose(0, 1, 3, 5, 2, 4)
         .reshape(B, H, W, C))
  return out


# speedup vs baseline: 1.1071x; 1.1071x over previous
import functools

import jax
import jax.numpy as jnp
from jax import lax
from jax.experimental import pallas as pl
from jax.experimental.pallas import tpu as pltpu
from jax.experimental.pallas import tpu_sc as plsc

N_WORKERS = 32
BLOCKS_PER_CHUNK = 2
IDX_PER_CHUNK = BLOCKS_PER_CHUNK * 8 * 128
NBUF = 2


def _render(idx_blk, feats, n_blocks, n_bh):
  C = feats.shape[1]
  blocks_per_w = n_blocks // N_WORKERS
  chunks_per_w = blocks_per_w // BLOCKS_PER_CHUNK

  mesh = plsc.VectorSubcoreMesh(core_axis_name="c", subcore_axis_name="s")

  @functools.partial(
      pl.kernel,
      mesh=mesh,
      out_type=jax.ShapeDtypeStruct((n_bh, 2, 4, 1024), jnp.float32),
      compiler_params=pltpu.CompilerParams(
          use_tc_tiling_on_sc=False, needs_layout_passes=False),
      scratch_types=(
          [pltpu.VMEM((2 * BLOCKS_PER_CHUNK, 8, 128), jnp.int32)] * NBUF
          + [pltpu.VMEM((IDX_PER_CHUNK, C), jnp.float32)] * NBUF
          + [pltpu.VMEM((BLOCKS_PER_CHUNK * 16 * 128,), jnp.float32)] * NBUF
          + [pltpu.SemaphoreType.DMA] * (3 * NBUF)
      ),
  )
  def k(idx_hbm, feat_hbm, out_hbm, idx_v0, idx_v1, rows_v0, rows_v1,
        out_v0, out_v1, sem0, sem1, isem0, isem1, osem0, osem1):
    wid = lax.axis_index("s") * 2 + lax.axis_index("c")
    blk0 = wid * blocks_per_w
    idx_bufs = (idx_v0, idx_v1)
    rows_bufs = (rows_v0, rows_v1)
    out_bufs = (out_v0, out_v1)
    sems = (sem0, sem1)
    isems = (isem0, isem1)
    osems = (osem0, osem1)
    iota = lax.iota(jnp.int32, 16)

    def idx_copy(g2, islot):
      pltpu.async_copy(
          idx_hbm.at[pl.ds(blk0 + g2 * 2 * BLOCKS_PER_CHUNK,
                           2 * BLOCKS_PER_CHUNK)],
          idx_bufs[islot], isems[islot])

    def idx_wait(g2, islot):
      pltpu.make_async_copy(
          idx_hbm.at[pl.ds(blk0 + g2 * 2 * BLOCKS_PER_CHUNK,
                           2 * BLOCKS_PER_CHUNK)],
          idx_bufs[islot], isems[islot]).wait()

    def fire(g, slot, islot, half):
      idx_v = idx_bufs[islot]
      rows_v = rows_bufs[slot]
      for b2 in range(BLOCKS_PER_CHUNK):
        for kk in range(8):
          pltpu.async_copy(
              feat_hbm.at[idx_v.at[half * BLOCKS_PER_CHUNK + b2, kk]],
              rows_v.at[pl.ds((b2 * 8 + kk) * 128, 128)],
              sems[slot])

    def drain(slot, islot, half):
      for b2 in range(BLOCKS_PER_CHUNK):
        for kk in range(8):
          pltpu.make_async_copy(
              feat_hbm.at[idx_bufs[islot].at[half * BLOCKS_PER_CHUNK + b2,
                                             kk]],
              rows_bufs[slot].at[pl.ds((b2 * 8 + kk) * 128, 128)],
              sems[slot]).wait()

    def out_slices(g, slot):
      pairs = []
      for b2 in range(BLOCKS_PER_CHUNK):
        blk = blk0 + g * BLOCKS_PER_CHUNK + b2
        bh = blk // 4
        wt = blk % 4
        for ct in range(2):
          pairs.append((
              out_bufs[slot].at[pl.ds(b2 * 2048 + ct * 1024, 1024)],
              out_hbm.at[bh, ct, wt]))
      return pairs

    def out_drain(g, slot):
      for src, dst in out_slices(g, slot):
        pltpu.make_async_copy(src, dst, osems[slot]).wait()

    def reduce_store(g, slot):
      rows_v = rows_bufs[slot]
      out_v = out_bufs[slot]

      @pl.when(g >= 2)
      def _():
        out_drain(g, slot)

      def tree(l):
        s0 = l[0] + l[1]
        s1 = l[2] + l[3]
        s2 = l[4] + l[5]
        s3 = l[6] + l[7]
        return ((s0 + s1) + (s2 + s3)) * 0.125

      for b2 in range(BLOCKS_PER_CHUNK):
        c_off = iota * 128 + b2 * 2048

        def body(w, _b2=b2, _c_off=c_off):
          P = 4
          loads = []
          accs = [None] * P
          for j in range(P):
            base = _b2 * 1024 + w + j
            loads.append([rows_v[base + 128 * kk] for kk in range(8)])
            if j >= 1:
              accs[j - 1] = tree(loads[j - 1])
          accs[P - 1] = tree(loads[P - 1])
          for j in range(P):
            plsc.store_scatter(out_v, [_c_off + (w + j)], accs[j])

        plsc.parallel_loop(0, 128, 4, unroll=2)(body)

      for src, dst in out_slices(g, slot):
        pltpu.async_copy(src, dst, osems[slot])

    n_pairs = chunks_per_w // 2
    idx_copy(0, 0)
    idx_wait(0, 0)
    idx_copy(1, 1)
    fire(0, 0, 0, 0)

    def step(g4, _):
      g = g4 * 4
      fire(g + 1, 1, 0, 1)
      drain(0, 0, 0)
      reduce_store(g, 0)
      idx_wait(2 * g4 + 1, 1)
      fire(g + 2, 0, 1, 0)
      drain(1, 0, 1)
      reduce_store(g + 1, 1)

      @pl.when(2 * g4 + 2 < n_pairs)
      def _():
        idx_copy(2 * g4 + 2, 0)

      fire(g + 3, 1, 1, 1)
      drain(0, 1, 0)
      reduce_store(g + 2, 0)

      @pl.when(g + 4 < chunks_per_w)
      def _():
        idx_wait(2 * g4 + 2, 0)
        fire(g + 4, 0, 0, 0)

      drain(1, 1, 1)
      reduce_store(g + 3, 1)

      @pl.when(2 * g4 + 3 < n_pairs)
      def _():
        idx_copy(2 * g4 + 3, 1)

      return ()

    lax.fori_loop(0, chunks_per_w // 4, step, ())
    out_drain(0, 0)
    out_drain(0, 1)

  return k(idx_blk, feats)


_FMT_PBLK = 4096


def _format_features(ftT, P, C):
  n_blk = (P + _FMT_PBLK - 1) // _FMT_PBLK
  out_rows = n_blk * (_FMT_PBLK * C // 128)

  def body(x_ref, o_ref):
    n = _FMT_PBLK * C // 128
    y3 = jnp.swapaxes(x_ref[...], 0, 1).reshape(n, 128 // C, C)
    o_ref[...] = jnp.concatenate(
        [y3[:, cs, :] for cs in range(128 // C)], axis=1)

  return pl.pallas_call(
      body,
      grid=(n_blk,),
      in_specs=[pl.BlockSpec((C, _FMT_PBLK), lambda i: (0, i))],
      out_specs=pl.BlockSpec((_FMT_PBLK * C // 128, 128), lambda i: (i, 0)),
      out_shape=jax.ShapeDtypeStruct((out_rows, 128), jnp.float32),
  )(ftT)


def kernel(idx, dists, features):
  del dists
  B, H, W, K = idx.shape
  P, C = features.shape
  n_bh = B * H
  n_blocks = n_bh * (W // 128)
  idx_blk = (idx.astype(jnp.int32)
             .reshape(B, H, W // 128, 128, K)
             .transpose(0, 1, 2, 4, 3)
             .reshape(n_blocks, K, 128))
  feats_lin = _format_features(features.T, P, C)
  feats_lin = feats_lin.reshape(feats_lin.shape[0] * 128 // C, C)
  out5 = _render(idx_blk, feats_lin, n_blocks, n_bh)
  out = (out5.reshape(B, H, 2, W // 128, 8, 128)
         .transpose(0, 1, 3, 5, 2, 4)
         .reshape(B, H, W, C))
  return out
